# Initial kernel scaffold; baseline (speedup 1.0000x reference)
import jax
import jax.numpy as jnp
from jax.experimental import pallas as pl


def kernel(x, pos, edge_index, batch, ec_params, gcn_params):
    raise NotImplementedError("write your pallas kernel here")


# speedup vs baseline: 3.2290x; 3.2290x over previous
import jax
import jax.numpy as jnp
from jax.experimental import pallas as pl

_K = 6
_TILE = 128
_MASKVAL = 1e30
_TAKEN = 1e32
_BIGIDX = 2**30


def _knn_kernel(feat_ref, featT_ref, brow_ref, bcol_ref, idx_ref):
    i = pl.program_id(0)
    feat = feat_ref[...]
    ft = featT_ref[...]
    sqc = jnp.sum(ft * ft, axis=0, keepdims=True)
    d = sqc - 2.0 * jnp.dot(feat, ft, preferred_element_type=jnp.float32)
    brow = brow_ref[...]
    bcol = bcol_ref[...]
    colv = jax.lax.broadcasted_iota(jnp.int32, (1, d.shape[1]), 1)
    rowv = i * _TILE + jax.lax.broadcasted_iota(jnp.int32, (_TILE, 1), 0)
    d = jnp.where((brow != bcol) | (rowv == colv), _MASKVAL, d)
    cols = []
    for _ in range(_K):
        m = jnp.min(d, axis=1, keepdims=True)
        a = jnp.min(jnp.where(d == m, colv, _BIGIDX), axis=1).astype(jnp.int32)
        cols.append(a)
        d = jnp.where(colv == a[:, None], _TAKEN, d)
    idx_ref[...] = jnp.stack(cols, axis=1)


def _knn(feat, batch):
    n, c = feat.shape
    npad = ((n + _TILE - 1) // _TILE) * _TILE
    dp = ((c + 7) // 8) * 8
    fp = jnp.zeros((npad, dp), jnp.float32).at[:n, :c].set(feat)
    bp = jnp.full((npad,), -1, jnp.int32).at[:n].set(batch)
    idx = pl.pallas_call(
        _knn_kernel,
        grid=(npad // _TILE,),
        in_specs=[
            pl.BlockSpec((_TILE, dp), lambda i: (i, 0)),
            pl.BlockSpec((dp, npad), lambda i: (0, 0)),
            pl.BlockSpec((_TILE, 1), lambda i: (i, 0)),
            pl.BlockSpec((1, npad), lambda i: (0, 0)),
        ],
        out_specs=pl.BlockSpec((_TILE, _K), lambda i: (i, 0)),
        out_shape=jax.ShapeDtypeStruct((npad, _K), jnp.int32),
    )(fp, fp.T, bp[:, None], bp[None, :])
    return idx[:n]


def _leaky(v):
    return jnp.where(v >= 0, v, 0.1 * v)


def _bnorm(v, g, b):
    m = jnp.mean(v, axis=0)
    var = jnp.var(v, axis=0)
    return (v - m) / jnp.sqrt(var + 1e-5) * g + b


def _edge_conv(feat, batch, p):
    idx = _knn(feat, batch)
    xj = feat[idx]
    xi = jnp.broadcast_to(feat[:, None, :], xj.shape)
    tmp = jnp.concatenate([xi, xj - xi], axis=-1).reshape(-1, 2 * feat.shape[1])
    h = _leaky(tmp @ p["W1"] + p["b1"])
    h = _bnorm(h, p["g1"], p["be1"])
    h = _leaky(h @ p["W2"] + p["b2"])
    h = _bnorm(h, p["g2"], p["be2"])
    h = h @ p["W3"] + p["b3"]
    return h.reshape(feat.shape[0], _K, -1).sum(axis=1)


def _gcn_layer(feat, edge_index, p):
    n = feat.shape[0]
    loops = jnp.arange(n, dtype=edge_index.dtype)
    src = jnp.concatenate([edge_index[0], loops])
    dst = jnp.concatenate([edge_index[1], loops])
    deg = jnp.zeros((n,), jnp.float32).at[dst].add(1.0)
    dinv = jnp.where(deg > 0, 1.0 / jnp.sqrt(deg), 0.0)
    norm = dinv[src] * dinv[dst]
    h = feat @ p["W"]
    out = jnp.zeros((n, h.shape[1]), h.dtype).at[dst].add(h[src] * norm[:, None])
    return out + p["b"]


def kernel(x, pos, edge_index, batch, ec_params, gcn_params):
    h = _edge_conv(pos, batch, ec_params[0])
    append = h
    for p in ec_params[1:3]:
        h = _edge_conv(h, batch, p)
    h = _edge_conv(h, batch, ec_params[3])
    err = _edge_conv(jnp.concatenate([append, h], axis=1), batch, ec_params[4])
    u = _leaky(_gcn_layer(jnp.concatenate([x, err], axis=1), edge_index, gcn_params[0]))
    for p in gcn_params[1:4]:
        u = _leaky(_gcn_layer(u, edge_index, p))
    u = _gcn_layer(u, edge_index, gcn_params[4])
    return u
